# Initial kernel scaffold; baseline (speedup 1.0000x reference)
import jax
import jax.numpy as jnp
from jax.experimental import pallas as pl


def kernel(z, W_in, b_in, W_out, b_out, codebooks):
    raise NotImplementedError("write your pallas kernel here")


# speedup vs baseline: 2.6191x; 2.6191x over previous
import jax
import jax.numpy as jnp
from jax.experimental import pallas as pl
from jax.experimental.pallas import tpu as pltpu


def _rvq_body(z_ref, wi_ref, bi_ref, wo_ref, bo_ref, cb_ref,
              zq_ref, codes_ref, lat_ref, loss_ref):
    n_cb, cdim, _ = wi_ref.shape
    k = cb_ref.shape[1]
    t_blk = z_ref.shape[2]

    resid = z_ref[0]
    zq_acc = jnp.zeros_like(resid)
    loss = jnp.float32(0.0)
    iota_k = jax.lax.broadcasted_iota(jnp.int32, (k, t_blk), 0)

    for i in range(n_cb):
        wi = wi_ref[i]
        bi = bi_ref[i]
        wo = wo_ref[i]
        bo = bo_ref[i]
        cb = cb_ref[i]

        z_e = jnp.dot(wi, resid, preferred_element_type=jnp.float32) + bi[:, None]

        en = jnp.sqrt(jnp.sum(z_e * z_e, axis=0, keepdims=True))
        z_e_n = z_e / jnp.maximum(en, 1e-12)
        a = jnp.sum(z_e_n * z_e_n, axis=0, keepdims=True)
        cbn = jnp.sqrt(jnp.sum(cb * cb, axis=1, keepdims=True))
        cb_n = cb / jnp.maximum(cbn, 1e-12)
        c = jnp.sum(cb_n * cb_n, axis=1, keepdims=True)

        d = jax.lax.dot_general(cb_n, z_e_n, (((1,), (0,)), ((), ())),
                                preferred_element_type=jnp.float32)
        neg_dist = -((a - 2.0 * d) + c)

        m = jnp.max(neg_dist, axis=0, keepdims=True)
        idx = jnp.min(jnp.where(neg_dist == m, iota_k, k), axis=0)

        oh = (iota_k == idx[None, :]).astype(jnp.float32)
        z_q = jax.lax.dot_general(cb, oh, (((0,), (0,)), ((), ())),
                                  preferred_element_type=jnp.float32)

        diff = z_e - z_q
        loss = loss + jnp.sum(diff * diff)

        out = jnp.dot(wo, z_q, preferred_element_type=jnp.float32) + bo[:, None]
        zq_acc = zq_acc + out
        resid = resid - out

        codes_ref[0, i, :] = idx
        lat_ref[0, i * cdim:(i + 1) * cdim, :] = z_e

    zq_ref[0] = zq_acc
    loss_ref[0, 0, :] = jnp.broadcast_to(loss, (loss_ref.shape[2],))


def kernel(z, W_in, b_in, W_out, b_out, codebooks):
    b, dmodel, t = z.shape
    n_cb, cdim, _ = W_in.shape

    t_blk = 512 if t % 512 == 0 else t
    gt = t // t_blk
    grid = (b, gt)

    full = lambda shape: pl.BlockSpec(shape, lambda bi, ti: (0,) * len(shape))

    zq, codes, lat, losses = pl.pallas_call(
        _rvq_body,
        grid=grid,
        in_specs=[
            pl.BlockSpec((1, dmodel, t_blk), lambda bi, ti: (bi, 0, ti)),
            full(W_in.shape),
            full(b_in.shape),
            full(W_out.shape),
            full(b_out.shape),
            full(codebooks.shape),
        ],
        out_specs=[
            pl.BlockSpec((1, dmodel, t_blk), lambda bi, ti: (bi, 0, ti)),
            pl.BlockSpec((1, n_cb, t_blk), lambda bi, ti: (bi, 0, ti)),
            pl.BlockSpec((1, n_cb * cdim, t_blk), lambda bi, ti: (bi, 0, ti)),
            pl.BlockSpec((1, 1, 128), lambda bi, ti: (bi * gt + ti, 0, 0)),
        ],
        out_shape=[
            jax.ShapeDtypeStruct((b, dmodel, t), jnp.float32),
            jax.ShapeDtypeStruct((b, n_cb, t), jnp.int32),
            jax.ShapeDtypeStruct((b, n_cb * cdim, t), jnp.float32),
            jax.ShapeDtypeStruct((b * gt, 1, 128), jnp.float32),
        ],
        compiler_params=pltpu.CompilerParams(
            dimension_semantics=("parallel", "parallel")),
    )(z, W_in, b_in, W_out, b_out, codebooks)

    total = jnp.sum(losses[:, 0, 0])
    commitment_loss = total / jnp.float32(b * cdim * t)
    codebook_loss = commitment_loss
    return zq, codes, lat, commitment_loss, codebook_loss


# speedup vs baseline: 3.6490x; 1.3932x over previous
import jax
import jax.numpy as jnp
from jax.experimental import pallas as pl
from jax.experimental.pallas import tpu as pltpu


def _rvq_body(z_ref, wi_ref, bi_ref, wo_ref, bo_ref, cb_ref,
              zq_ref, codes_ref, lat_ref, loss_ref):
    n_cb, cdim, _ = wi_ref.shape
    k = cb_ref.shape[1]
    t_blk = z_ref.shape[2]

    resid = z_ref[0]
    loss = jnp.float32(0.0)
    iota_k = jax.lax.broadcasted_iota(jnp.int32, (k, t_blk), 0)
    ones_row = jnp.ones((1, t_blk), jnp.float32)

    for i in range(n_cb):
        wi = wi_ref[i]
        bi = bi_ref[i]
        wo = wo_ref[i]
        bo = bo_ref[i]
        cb = cb_ref[i]

        z_e = jnp.dot(wi, resid, preferred_element_type=jnp.float32) + bi[:, None]

        en = jnp.sqrt(jnp.sum(z_e * z_e, axis=0, keepdims=True))
        z_e_n = z_e / jnp.maximum(en, 1e-12)
        a = jnp.sum(z_e_n * z_e_n, axis=0, keepdims=True)
        cbn = jnp.sqrt(jnp.sum(cb * cb, axis=1, keepdims=True))
        cb_n = cb / jnp.maximum(cbn, 1e-12)
        c = jnp.sum(cb_n * cb_n, axis=1, keepdims=True)

        d = jax.lax.dot_general(cb_n, z_e_n, (((1,), (0,)), ((), ())),
                                preferred_element_type=jnp.float32)
        neg_dist = -((a - 2.0 * d) + c)

        m = jnp.max(neg_dist, axis=0, keepdims=True)
        idx = jnp.min(jnp.where(neg_dist == m, iota_k, k), axis=0)

        oh = (iota_k == idx[None, :]).astype(jnp.float32)
        z_q = jax.lax.dot_general(cb, oh, (((0,), (0,)), ((), ())),
                                  preferred_element_type=jnp.float32)

        diff = z_e - z_q
        loss = loss + jnp.sum(diff * diff)

        wo_aug = jnp.concatenate([wo, bo[:, None]], axis=1)
        zq_aug = jnp.concatenate([z_q, ones_row], axis=0)
        out = jnp.dot(wo_aug, zq_aug, preferred_element_type=jnp.float32)
        resid = resid - out

        codes_ref[0, i, :] = idx
        lat_ref[0, i * cdim:(i + 1) * cdim, :] = z_e

    zq_ref[0] = z_ref[0] - resid
    loss_ref[0, 0, :] = jnp.broadcast_to(loss, (loss_ref.shape[2],))


def kernel(z, W_in, b_in, W_out, b_out, codebooks):
    b, dmodel, t = z.shape
    n_cb, cdim, _ = W_in.shape

    t_blk = 1024 if t % 1024 == 0 else t
    gt = t // t_blk
    grid = (b, gt)

    full = lambda shape: pl.BlockSpec(shape, lambda bi, ti: (0,) * len(shape))

    zq, codes, lat, losses = pl.pallas_call(
        _rvq_body,
        grid=grid,
        in_specs=[
            pl.BlockSpec((1, dmodel, t_blk), lambda bi, ti: (bi, 0, ti)),
            full(W_in.shape),
            full(b_in.shape),
            full(W_out.shape),
            full(b_out.shape),
            full(codebooks.shape),
        ],
        out_specs=[
            pl.BlockSpec((1, dmodel, t_blk), lambda bi, ti: (bi, 0, ti)),
            pl.BlockSpec((1, n_cb, t_blk), lambda bi, ti: (bi, 0, ti)),
            pl.BlockSpec((1, n_cb * cdim, t_blk), lambda bi, ti: (bi, 0, ti)),
            pl.BlockSpec((1, 1, 128), lambda bi, ti: (bi * gt + ti, 0, 0)),
        ],
        out_shape=[
            jax.ShapeDtypeStruct((b, dmodel, t), jnp.float32),
            jax.ShapeDtypeStruct((b, n_cb, t), jnp.int32),
            jax.ShapeDtypeStruct((b, n_cb * cdim, t), jnp.float32),
            jax.ShapeDtypeStruct((b * gt, 1, 128), jnp.float32),
        ],
        compiler_params=pltpu.CompilerParams(
            dimension_semantics=("parallel", "parallel")),
    )(z, W_in, b_in, W_out, b_out, codebooks)

    total = jnp.sum(losses[:, 0, 0])
    commitment_loss = total / jnp.float32(b * cdim * t)
    codebook_loss = commitment_loss
    return zq, codes, lat, commitment_loss, codebook_loss


# speedup vs baseline: 4.7872x; 1.3119x over previous
import jax
import jax.numpy as jnp
from jax.experimental import pallas as pl
from jax.experimental.pallas import tpu as pltpu


def _prep_body(cb_ref, aug_ref):
    n_cb, k, cdim = cb_ref.shape
    cb = cb_ref[...]
    nrm = jnp.sqrt(jnp.sum(cb * cb, axis=2, keepdims=True))
    cb_n = cb / jnp.maximum(nrm, 1e-12)
    c = jnp.sum(cb_n * cb_n, axis=2, keepdims=True)
    pad = jnp.zeros((n_cb, k, aug_ref.shape[2] - cdim - 1), jnp.float32)
    aug_ref[...] = jnp.concatenate([2.0 * cb_n, -c, pad], axis=2)


def _rvq_body(z_ref, wi_ref, bi_ref, wo_ref, bo_ref, cb_ref, aug_ref,
              zq_ref, codes_ref, lat_ref, loss_ref):
    n_cb, cdim, _ = wi_ref.shape
    k = cb_ref.shape[1]
    t_blk = z_ref.shape[2]
    caug = aug_ref.shape[2]

    resid = z_ref[0]
    loss = jnp.float32(0.0)
    iota_k = jax.lax.broadcasted_iota(jnp.int32, (k, t_blk), 0)
    ones_row = jnp.ones((1, t_blk), jnp.float32)
    zero_rows = jnp.zeros((caug - cdim - 1, t_blk), jnp.float32)

    for i in range(n_cb):
        wi = wi_ref[i]
        bi = bi_ref[i]
        wo = wo_ref[i]
        bo = bo_ref[i]
        cb = cb_ref[i]
        aug = aug_ref[i]

        z_e = jnp.dot(wi, resid, preferred_element_type=jnp.float32) + bi[:, None]

        en = jnp.sqrt(jnp.sum(z_e * z_e, axis=0, keepdims=True))
        z_e_n = z_e / jnp.maximum(en, 1e-12)

        z_aug = jnp.concatenate([z_e_n, ones_row, zero_rows], axis=0)
        score = jax.lax.dot_general(aug, z_aug, (((1,), (0,)), ((), ())),
                                    preferred_element_type=jnp.float32)

        m = jnp.max(score, axis=0, keepdims=True)
        idx = jnp.min(jnp.where(score == m, iota_k, k), axis=0)

        oh = (iota_k == idx[None, :]).astype(jnp.float32)
        z_q = jax.lax.dot_general(cb, oh, (((0,), (0,)), ((), ())),
                                  preferred_element_type=jnp.float32)

        diff = z_e - z_q
        loss = loss + jnp.sum(diff * diff)

        wo_aug = jnp.concatenate([wo, bo[:, None]], axis=1)
        zq_aug = jnp.concatenate([z_q, ones_row], axis=0)
        out = jnp.dot(wo_aug, zq_aug, preferred_element_type=jnp.float32)
        resid = resid - out

        codes_ref[0, i, :] = idx
        lat_ref[0, i * cdim:(i + 1) * cdim, :] = z_e

    zq_ref[0] = z_ref[0] - resid
    loss_ref[0, 0, :] = jnp.broadcast_to(loss, (loss_ref.shape[2],))


def kernel(z, W_in, b_in, W_out, b_out, codebooks):
    b, dmodel, t = z.shape
    n_cb, cdim, _ = W_in.shape
    kk = codebooks.shape[1]
    caug = 16

    t_blk = 1024 if t % 1024 == 0 else t
    gt = t // t_blk
    grid = (b, gt)

    full = lambda shape: pl.BlockSpec(shape, lambda *_: (0,) * len(shape))

    cb_aug = pl.pallas_call(
        _prep_body,
        out_shape=jax.ShapeDtypeStruct((n_cb, kk, caug), jnp.float32),
    )(codebooks)

    zq, codes, lat, losses = pl.pallas_call(
        _rvq_body,
        grid=grid,
        in_specs=[
            pl.BlockSpec((1, dmodel, t_blk), lambda bi, ti: (bi, 0, ti)),
            full(W_in.shape),
            full(b_in.shape),
            full(W_out.shape),
            full(b_out.shape),
            full(codebooks.shape),
            full((n_cb, kk, caug)),
        ],
        out_specs=[
            pl.BlockSpec((1, dmodel, t_blk), lambda bi, ti: (bi, 0, ti)),
            pl.BlockSpec((1, n_cb, t_blk), lambda bi, ti: (bi, 0, ti)),
            pl.BlockSpec((1, n_cb * cdim, t_blk), lambda bi, ti: (bi, 0, ti)),
            pl.BlockSpec((1, 1, 128), lambda bi, ti: (bi * gt + ti, 0, 0)),
        ],
        out_shape=[
            jax.ShapeDtypeStruct((b, dmodel, t), jnp.float32),
            jax.ShapeDtypeStruct((b, n_cb, t), jnp.int32),
            jax.ShapeDtypeStruct((b, n_cb * cdim, t), jnp.float32),
            jax.ShapeDtypeStruct((b * gt, 1, 128), jnp.float32),
        ],
        compiler_params=pltpu.CompilerParams(
            dimension_semantics=("parallel", "parallel")),
    )(z, W_in, b_in, W_out, b_out, codebooks, cb_aug)

    total = jnp.sum(losses[:, 0, 0])
    commitment_loss = total / jnp.float32(b * cdim * t)
    codebook_loss = commitment_loss
    return zq, codes, lat, commitment_loss, codebook_loss


# speedup vs baseline: 4.7878x; 1.0001x over previous
import jax
import jax.numpy as jnp
from jax.experimental import pallas as pl
from jax.experimental.pallas import tpu as pltpu


def _prep_body(cb_ref, aug_ref):
    n_cb, k, cdim = cb_ref.shape
    cb = cb_ref[...]
    nrm = jnp.sqrt(jnp.sum(cb * cb, axis=2, keepdims=True))
    cb_n = cb / jnp.maximum(nrm, 1e-12)
    c = jnp.sum(cb_n * cb_n, axis=2, keepdims=True)
    pad = jnp.zeros((n_cb, k, aug_ref.shape[2] - cdim - 1), jnp.float32)
    aug_ref[...] = jnp.concatenate([2.0 * cb_n, -c, pad], axis=2)


def _rvq_body(z_ref, wi_ref, bi_ref, wo_ref, bo_ref, cb_ref, aug_ref,
              zq_ref, codes_ref, lat_ref, loss_ref):
    n_cb, cdim, _ = wi_ref.shape
    k = cb_ref.shape[1]
    t_blk = z_ref.shape[2]
    caug = aug_ref.shape[2]

    resid = z_ref[0]
    loss = jnp.float32(0.0)
    iota_k = jax.lax.broadcasted_iota(jnp.int32, (k, t_blk), 0)
    ones_row = jnp.ones((1, t_blk), jnp.float32)
    zero_rows = jnp.zeros((caug - cdim - 1, t_blk), jnp.float32)

    for i in range(n_cb):
        wi = wi_ref[i]
        bi = bi_ref[i]
        wo = wo_ref[i]
        bo = bo_ref[i]
        cb = cb_ref[i]
        aug = aug_ref[i]

        z_e = jnp.dot(wi, resid, preferred_element_type=jnp.float32) + bi[:, None]

        en = jnp.sqrt(jnp.sum(z_e * z_e, axis=0, keepdims=True))
        z_e_n = z_e / jnp.maximum(en, 1e-12)

        z_aug = jnp.concatenate([z_e_n, ones_row, zero_rows], axis=0)
        score = jax.lax.dot_general(aug, z_aug, (((1,), (0,)), ((), ())),
                                    preferred_element_type=jnp.float32)

        m = jnp.max(score, axis=0, keepdims=True)
        idx = jnp.min(jnp.where(score == m, iota_k, k), axis=0)

        oh = (iota_k == idx[None, :]).astype(jnp.float32)
        z_q = jax.lax.dot_general(cb, oh, (((0,), (0,)), ((), ())),
                                  preferred_element_type=jnp.float32)

        diff = z_e - z_q
        loss = loss + jnp.sum(diff * diff)

        z_q_st = z_e + (z_q - z_e)

        wo_aug = jnp.concatenate([wo, bo[:, None]], axis=1)
        zq_aug = jnp.concatenate([z_q_st, ones_row], axis=0)
        out = jnp.dot(wo_aug, zq_aug, preferred_element_type=jnp.float32)
        resid = resid - out

        codes_ref[0, i, :] = idx
        lat_ref[0, i * cdim:(i + 1) * cdim, :] = z_e

    zq_ref[0] = z_ref[0] - resid
    loss_ref[0, 0, :] = jnp.broadcast_to(loss, (loss_ref.shape[2],))


def kernel(z, W_in, b_in, W_out, b_out, codebooks):
    b, dmodel, t = z.shape
    n_cb, cdim, _ = W_in.shape
    kk = codebooks.shape[1]
    caug = 16

    t_blk = 1024 if t % 1024 == 0 else t
    gt = t // t_blk
    grid = (b, gt)

    full = lambda shape: pl.BlockSpec(shape, lambda *_: (0,) * len(shape))

    cb_aug = pl.pallas_call(
        _prep_body,
        out_shape=jax.ShapeDtypeStruct((n_cb, kk, caug), jnp.float32),
    )(codebooks)

    zq, codes, lat, losses = pl.pallas_call(
        _rvq_body,
        grid=grid,
        in_specs=[
            pl.BlockSpec((1, dmodel, t_blk), lambda bi, ti: (bi, 0, ti)),
            full(W_in.shape),
            full(b_in.shape),
            full(W_out.shape),
            full(b_out.shape),
            full(codebooks.shape),
            full((n_cb, kk, caug)),
        ],
        out_specs=[
            pl.BlockSpec((1, dmodel, t_blk), lambda bi, ti: (bi, 0, ti)),
            pl.BlockSpec((1, n_cb, t_blk), lambda bi, ti: (bi, 0, ti)),
            pl.BlockSpec((1, n_cb * cdim, t_blk), lambda bi, ti: (bi, 0, ti)),
            pl.BlockSpec((1, 1, 128), lambda bi, ti: (bi * gt + ti, 0, 0)),
        ],
        out_shape=[
            jax.ShapeDtypeStruct((b, dmodel, t), jnp.float32),
            jax.ShapeDtypeStruct((b, n_cb, t), jnp.int32),
            jax.ShapeDtypeStruct((b, n_cb * cdim, t), jnp.float32),
            jax.ShapeDtypeStruct((b * gt, 1, 128), jnp.float32),
        ],
        compiler_params=pltpu.CompilerParams(
            dimension_semantics=("parallel", "parallel")),
    )(z, W_in, b_in, W_out, b_out, codebooks, cb_aug)

    total = jnp.sum(losses[:, 0, 0])
    commitment_loss = total / jnp.float32(b * cdim * t)
    codebook_loss = commitment_loss
    return zq, codes, lat, commitment_loss, codebook_loss


# speedup vs baseline: 5.1521x; 1.0761x over previous
import jax
import jax.numpy as jnp
from jax.experimental import pallas as pl
from jax.experimental.pallas import tpu as pltpu


def _prep_body(cb_ref, aug_ref):
    n_cb, k, cdim = cb_ref.shape
    cb = cb_ref[...]
    nrm = jnp.sqrt(jnp.sum(cb * cb, axis=2, keepdims=True))
    cb_n = cb / jnp.maximum(nrm, 1e-12)
    c = jnp.sum(cb_n * cb_n, axis=2, keepdims=True)
    pad = jnp.zeros((n_cb, k, aug_ref.shape[2] - cdim - 1), jnp.float32)
    aug_ref[...] = jnp.concatenate([2.0 * cb_n, -c, pad], axis=2)


def _rvq_body(z_ref, wi_ref, bi_ref, wo_ref, bo_ref, cbr_ref, aug_ref,
              zq_ref, codes_ref, lat_ref, loss_ref):
    n_cb, cdim, _ = wi_ref.shape
    k = aug_ref.shape[1]
    k_lo = cbr_ref.shape[1]
    n_hi = k // k_lo
    t_blk = z_ref.shape[2]
    caug = aug_ref.shape[2]

    resid = z_ref[0]
    loss = jnp.float32(0.0)
    iota_k = jax.lax.broadcasted_iota(jnp.int32, (k, t_blk), 0)
    iota_lo = jax.lax.broadcasted_iota(jnp.int32, (k_lo, t_blk), 0)
    iota_hi = jax.lax.broadcasted_iota(jnp.int32, (n_hi * cdim, t_blk), 0) // cdim
    ones_row = jnp.ones((1, t_blk), jnp.float32)
    zero_rows = jnp.zeros((caug - cdim - 1, t_blk), jnp.float32)

    for i in range(n_cb):
        wi = wi_ref[i]
        bi = bi_ref[i]
        wo = wo_ref[i]
        bo = bo_ref[i]
        cbr = cbr_ref[i]
        aug = aug_ref[i]

        z_e = jnp.dot(wi, resid, preferred_element_type=jnp.float32) + bi[:, None]

        en = jnp.sqrt(jnp.sum(z_e * z_e, axis=0, keepdims=True))
        z_e_n = z_e / jnp.maximum(en, 1e-12)

        z_aug = jnp.concatenate([z_e_n, ones_row, zero_rows], axis=0)
        score = jax.lax.dot_general(aug, z_aug, (((1,), (0,)), ((), ())),
                                    preferred_element_type=jnp.float32)

        m = jnp.max(score, axis=0, keepdims=True)
        idx = jnp.min(jnp.where(score == m, iota_k, k), axis=0)

        lo = jnp.bitwise_and(idx, k_lo - 1)[None, :]
        hi = jnp.right_shift(idx, k_lo.bit_length() - 1)[None, :]
        oh_lo = (iota_lo == lo).astype(jnp.float32)
        cand = jax.lax.dot_general(cbr, oh_lo, (((0,), (0,)), ((), ())),
                                   preferred_element_type=jnp.float32)
        masked = jnp.where(iota_hi == hi, cand, 0.0)
        z_q = jnp.sum(masked.reshape(n_hi, cdim, t_blk), axis=0)

        diff = z_e - z_q
        loss = loss + jnp.sum(diff * diff)

        z_q_st = z_e + (z_q - z_e)

        wo_aug = jnp.concatenate([wo, bo[:, None]], axis=1)
        zq_aug = jnp.concatenate([z_q_st, ones_row], axis=0)
        out = jnp.dot(wo_aug, zq_aug, preferred_element_type=jnp.float32)
        resid = resid - out

        codes_ref[0, i, :] = idx
        lat_ref[0, i * cdim:(i + 1) * cdim, :] = z_e

    zq_ref[0] = z_ref[0] - resid
    loss_ref[0, 0, :] = jnp.broadcast_to(loss, (loss_ref.shape[2],))


def kernel(z, W_in, b_in, W_out, b_out, codebooks):
    b, dmodel, t = z.shape
    n_cb, cdim, _ = W_in.shape
    kk = codebooks.shape[1]
    caug = 16

    t_blk = 1024 if t % 1024 == 0 else t
    gt = t // t_blk
    grid = (b, gt)

    full = lambda shape: pl.BlockSpec(shape, lambda *_: (0,) * len(shape))

    cb_aug = pl.pallas_call(
        _prep_body,
        out_shape=jax.ShapeDtypeStruct((n_cb, kk, caug), jnp.float32),
    )(codebooks)

    k_lo = 128
    n_hi = kk // k_lo
    cb_resh = codebooks.reshape(n_cb, n_hi, k_lo, cdim)
    cb_resh = cb_resh.transpose(0, 2, 1, 3).reshape(n_cb, k_lo, n_hi * cdim)

    zq, codes, lat, losses = pl.pallas_call(
        _rvq_body,
        grid=grid,
        in_specs=[
            pl.BlockSpec((1, dmodel, t_blk), lambda bi, ti: (bi, 0, ti)),
            full(W_in.shape),
            full(b_in.shape),
            full(W_out.shape),
            full(b_out.shape),
            full((n_cb, k_lo, n_hi * cdim)),
            full((n_cb, kk, caug)),
        ],
        out_specs=[
            pl.BlockSpec((1, dmodel, t_blk), lambda bi, ti: (bi, 0, ti)),
            pl.BlockSpec((1, n_cb, t_blk), lambda bi, ti: (bi, 0, ti)),
            pl.BlockSpec((1, n_cb * cdim, t_blk), lambda bi, ti: (bi, 0, ti)),
            pl.BlockSpec((1, 1, 128), lambda bi, ti: (bi * gt + ti, 0, 0)),
        ],
        out_shape=[
            jax.ShapeDtypeStruct((b, dmodel, t), jnp.float32),
            jax.ShapeDtypeStruct((b, n_cb, t), jnp.int32),
            jax.ShapeDtypeStruct((b, n_cb * cdim, t), jnp.float32),
            jax.ShapeDtypeStruct((b * gt, 1, 128), jnp.float32),
        ],
        compiler_params=pltpu.CompilerParams(
            dimension_semantics=("parallel", "parallel")),
    )(z, W_in, b_in, W_out, b_out, cb_resh, cb_aug)

    total = jnp.sum(losses[:, 0, 0])
    commitment_loss = total / jnp.float32(b * cdim * t)
    codebook_loss = commitment_loss
    return zq, codes, lat, commitment_loss, codebook_loss


# speedup vs baseline: 5.5126x; 1.0700x over previous
import jax
import jax.numpy as jnp
from jax.experimental import pallas as pl
from jax.experimental.pallas import tpu as pltpu


def _prep_body(cb_ref, aug_ref):
    n_cb, k, cdim = cb_ref.shape
    cb = cb_ref[...]
    nrm = jnp.sqrt(jnp.sum(cb * cb, axis=2, keepdims=True))
    cb_n = cb / jnp.maximum(nrm, 1e-12)
    c = jnp.sum(cb_n * cb_n, axis=2, keepdims=True)
    pieces = [2.0 * cb_n, -c]
    if aug_ref.shape[2] > cdim + 1:
        pieces.append(jnp.zeros((n_cb, k, aug_ref.shape[2] - cdim - 1),
                                jnp.float32))
    aug_ref[...] = jnp.concatenate(pieces, axis=2)


def _rvq_body(z_ref, wi_ref, bi_ref, wo_ref, bo_ref, cbr_ref, aug_ref,
              zq_ref, codes_ref, lat_ref, loss_ref):
    n_cb, cdim, _ = wi_ref.shape
    k = aug_ref.shape[1]
    k_lo = cbr_ref.shape[1]
    n_hi = k // k_lo
    t_blk = z_ref.shape[2]
    caug = aug_ref.shape[2]

    resid = z_ref[0]
    loss = jnp.float32(0.0)
    iota_kf = jax.lax.broadcasted_iota(jnp.int32, (k, t_blk), 0).astype(jnp.float32)
    iota_lo = jax.lax.broadcasted_iota(jnp.int32, (k_lo, t_blk), 0)
    iota_hi = jax.lax.broadcasted_iota(jnp.int32, (n_hi * cdim, t_blk), 0) // cdim
    ones_row = jnp.ones((1, t_blk), jnp.float32)
    aug_pad = [jnp.zeros((caug - cdim - 1, t_blk), jnp.float32)] if caug > cdim + 1 else []

    for i in range(n_cb):
        wi = wi_ref[i]
        bi = bi_ref[i]
        wo = wo_ref[i]
        bo = bo_ref[i]
        cbr = cbr_ref[i]
        aug = aug_ref[i]

        z_e = jnp.dot(wi, resid, preferred_element_type=jnp.float32) + bi[:, None]

        en = jnp.sqrt(jnp.sum(z_e * z_e, axis=0, keepdims=True))
        z_e_n = z_e / jnp.maximum(en, 1e-12)

        z_aug = jnp.concatenate([z_e_n, ones_row] + aug_pad, axis=0)
        score = jax.lax.dot_general(aug, z_aug, (((1,), (0,)), ((), ())),
                                    preferred_element_type=jnp.float32)

        m = jnp.max(score, axis=0, keepdims=True)
        idx_f = jnp.min(jnp.where(score == m, iota_kf, jnp.float32(k)), axis=0)
        idx = idx_f.astype(jnp.int32)

        lo = jnp.bitwise_and(idx, k_lo - 1)[None, :]
        hi = jnp.right_shift(idx, k_lo.bit_length() - 1)[None, :]
        oh_lo = (iota_lo == lo).astype(jnp.float32)
        cand = jax.lax.dot_general(cbr, oh_lo, (((0,), (0,)), ((), ())),
                                   preferred_element_type=jnp.float32)
        masked = jnp.where(iota_hi == hi, cand, 0.0)
        z_q = jnp.sum(masked.reshape(n_hi, cdim, t_blk), axis=0)

        diff = z_e - z_q
        loss = loss + jnp.sum(diff * diff)

        z_q_st = z_e + (z_q - z_e)

        wo_aug = jnp.concatenate([wo, bo[:, None]], axis=1)
        zq_aug = jnp.concatenate([z_q_st, ones_row], axis=0)
        out = jnp.dot(wo_aug, zq_aug, preferred_element_type=jnp.float32)
        resid = resid - out

        codes_ref[0, i, :] = idx
        lat_ref[0, i * cdim:(i + 1) * cdim, :] = z_e

    zq_ref[0] = z_ref[0] - resid
    loss_ref[0, 0, :] = jnp.broadcast_to(loss, (loss_ref.shape[2],))


def kernel(z, W_in, b_in, W_out, b_out, codebooks):
    b, dmodel, t = z.shape
    n_cb, cdim, _ = W_in.shape
    kk = codebooks.shape[1]
    caug = cdim + 1

    t_blk = 1024 if t % 1024 == 0 else t
    gt = t // t_blk
    grid = (b, gt)

    full = lambda shape: pl.BlockSpec(shape, lambda *_: (0,) * len(shape))

    cb_aug = pl.pallas_call(
        _prep_body,
        out_shape=jax.ShapeDtypeStruct((n_cb, kk, caug), jnp.float32),
    )(codebooks)

    k_lo = 128
    n_hi = kk // k_lo
    cb_resh = codebooks.reshape(n_cb, n_hi, k_lo, cdim)
    cb_resh = cb_resh.transpose(0, 2, 1, 3).reshape(n_cb, k_lo, n_hi * cdim)

    zq, codes, lat, losses = pl.pallas_call(
        _rvq_body,
        grid=grid,
        in_specs=[
            pl.BlockSpec((1, dmodel, t_blk), lambda bi, ti: (bi, 0, ti)),
            full(W_in.shape),
            full(b_in.shape),
            full(W_out.shape),
            full(b_out.shape),
            full((n_cb, k_lo, n_hi * cdim)),
            full((n_cb, kk, caug)),
        ],
        out_specs=[
            pl.BlockSpec((1, dmodel, t_blk), lambda bi, ti: (bi, 0, ti)),
            pl.BlockSpec((1, n_cb, t_blk), lambda bi, ti: (bi, 0, ti)),
            pl.BlockSpec((1, n_cb * cdim, t_blk), lambda bi, ti: (bi, 0, ti)),
            pl.BlockSpec((1, 1, 128), lambda bi, ti: (bi * gt + ti, 0, 0)),
        ],
        out_shape=[
            jax.ShapeDtypeStruct((b, dmodel, t), jnp.float32),
            jax.ShapeDtypeStruct((b, n_cb, t), jnp.int32),
            jax.ShapeDtypeStruct((b, n_cb * cdim, t), jnp.float32),
            jax.ShapeDtypeStruct((b * gt, 1, 128), jnp.float32),
        ],
        compiler_params=pltpu.CompilerParams(
            dimension_semantics=("parallel", "parallel")),
    )(z, W_in, b_in, W_out, b_out, cb_resh, cb_aug)

    total = jnp.sum(losses[:, 0, 0])
    commitment_loss = total / jnp.float32(b * cdim * t)
    codebook_loss = commitment_loss
    return zq, codes, lat, commitment_loss, codebook_loss


# speedup vs baseline: 5.9964x; 1.0878x over previous
import jax
import jax.numpy as jnp
from jax.experimental import pallas as pl
from jax.experimental.pallas import tpu as pltpu


def _prep_body(cb_ref, aug_ref):
    n_cb, k, cdim = cb_ref.shape
    cb = cb_ref[...]
    nrm = jnp.sqrt(jnp.sum(cb * cb, axis=2, keepdims=True))
    cb_n = cb / jnp.maximum(nrm, 1e-12)
    c = jnp.sum(cb_n * cb_n, axis=2, keepdims=True)
    pieces = [2.0 * cb_n, -c]
    if aug_ref.shape[2] > cdim + 1:
        pieces.append(jnp.zeros((n_cb, k, aug_ref.shape[2] - cdim - 1),
                                jnp.float32))
    aug_ref[...] = jnp.concatenate(pieces, axis=2)


def _rvq_body(z_ref, wi_ref, bi_ref, wo_ref, bo_ref, cbr_ref, aug_ref,
              zq_ref, codes_ref, lat_ref, loss_ref):
    n_cb, cdim, _ = wi_ref.shape
    k = aug_ref.shape[1]
    k_lo = cbr_ref.shape[1]
    n_hi = k // k_lo
    t_blk = z_ref.shape[2]
    caug = aug_ref.shape[2]

    resid = z_ref[0]
    loss = jnp.float32(0.0)
    iota_kf = jax.lax.broadcasted_iota(jnp.int32, (k, t_blk), 0).astype(jnp.float32)
    iota_lo = jax.lax.broadcasted_iota(jnp.int32, (k_lo, t_blk), 0)
    iota_hi = jax.lax.broadcasted_iota(jnp.int32, (n_hi * cdim, t_blk), 0) // cdim
    ones_row = jnp.ones((1, t_blk), jnp.float32)
    aug_pad = [jnp.zeros((caug - cdim - 1, t_blk), jnp.float32)] if caug > cdim + 1 else []

    for i in range(n_cb):
        wi = wi_ref[i]
        bi = bi_ref[i]
        wo = wo_ref[i]
        bo = bo_ref[i]
        cbr = cbr_ref[i]
        aug = aug_ref[i]

        z_e = jnp.dot(wi, resid, preferred_element_type=jnp.float32) + bi[:, None]

        en = jnp.sqrt(jnp.sum(z_e * z_e, axis=0, keepdims=True))
        z_e_n = z_e / jnp.maximum(en, 1e-12)

        z_aug = jnp.concatenate([z_e_n, ones_row] + aug_pad, axis=0)
        score = jax.lax.dot_general(aug, z_aug, (((1,), (0,)), ((), ())),
                                    preferred_element_type=jnp.float32)

        m = jnp.max(score, axis=0, keepdims=True)
        idx_f = jnp.min(jnp.where(score == m, iota_kf, jnp.float32(k)), axis=0)
        idx = idx_f.astype(jnp.int32)

        lo = jnp.bitwise_and(idx, k_lo - 1)[None, :]
        hi = jnp.right_shift(idx, k_lo.bit_length() - 1)[None, :]
        oh_lo = (iota_lo == lo).astype(jnp.float32)
        cand = jax.lax.dot_general(cbr, oh_lo, (((0,), (0,)), ((), ())),
                                   preferred_element_type=jnp.float32)
        masked = jnp.where(iota_hi == hi, cand, 0.0)
        z_q = jnp.sum(masked.reshape(n_hi, cdim, t_blk), axis=0)

        diff = z_e - z_q
        loss = loss + jnp.sum(diff * diff)

        z_q_st = z_e + (z_q - z_e)

        wo_aug = jnp.concatenate([wo, bo[:, None]], axis=1)
        zq_aug = jnp.concatenate([z_q_st, ones_row], axis=0)
        out = jnp.dot(wo_aug, zq_aug, preferred_element_type=jnp.float32)
        resid = resid - out

        codes_ref[0, i, :] = idx
        lat_ref[0, i * cdim:(i + 1) * cdim, :] = z_e

    zq_ref[0] = z_ref[0] - resid
    loss_ref[0, 0, :] = jnp.broadcast_to(loss, (loss_ref.shape[2],))


def kernel(z, W_in, b_in, W_out, b_out, codebooks):
    b, dmodel, t = z.shape
    n_cb, cdim, _ = W_in.shape
    kk = codebooks.shape[1]
    caug = cdim + 1

    t_blk = 2048 if t % 2048 == 0 else t
    gt = t // t_blk
    grid = (b, gt)

    full = lambda shape: pl.BlockSpec(shape, lambda *_: (0,) * len(shape))

    cb_aug = pl.pallas_call(
        _prep_body,
        out_shape=jax.ShapeDtypeStruct((n_cb, kk, caug), jnp.float32),
    )(codebooks)

    k_lo = 128
    n_hi = kk // k_lo
    cb_resh = codebooks.reshape(n_cb, n_hi, k_lo, cdim)
    cb_resh = cb_resh.transpose(0, 2, 1, 3).reshape(n_cb, k_lo, n_hi * cdim)

    zq, codes, lat, losses = pl.pallas_call(
        _rvq_body,
        grid=grid,
        in_specs=[
            pl.BlockSpec((1, dmodel, t_blk), lambda bi, ti: (bi, 0, ti)),
            full(W_in.shape),
            full(b_in.shape),
            full(W_out.shape),
            full(b_out.shape),
            full((n_cb, k_lo, n_hi * cdim)),
            full((n_cb, kk, caug)),
        ],
        out_specs=[
            pl.BlockSpec((1, dmodel, t_blk), lambda bi, ti: (bi, 0, ti)),
            pl.BlockSpec((1, n_cb, t_blk), lambda bi, ti: (bi, 0, ti)),
            pl.BlockSpec((1, n_cb * cdim, t_blk), lambda bi, ti: (bi, 0, ti)),
            pl.BlockSpec((1, 1, 128), lambda bi, ti: (bi * gt + ti, 0, 0)),
        ],
        out_shape=[
            jax.ShapeDtypeStruct((b, dmodel, t), jnp.float32),
            jax.ShapeDtypeStruct((b, n_cb, t), jnp.int32),
            jax.ShapeDtypeStruct((b, n_cb * cdim, t), jnp.float32),
            jax.ShapeDtypeStruct((b * gt, 1, 128), jnp.float32),
        ],
        compiler_params=pltpu.CompilerParams(
            dimension_semantics=("parallel", "parallel")),
    )(z, W_in, b_in, W_out, b_out, cb_resh, cb_aug)

    total = jnp.sum(losses[:, 0, 0])
    commitment_loss = total / jnp.float32(b * cdim * t)
    codebook_loss = commitment_loss
    return zq, codes, lat, commitment_loss, codebook_loss
